# baseline (speedup 1.0000x reference)
import functools

import jax
import jax.numpy as jnp
from jax import lax
from jax.experimental import pallas as pl
from jax.experimental.pallas import tpu as pltpu
from jax.experimental.pallas import tpu_sc as plsc

CLS_NUM = 100000
FEATURE_DIM = 64
BATCH = 16384

_NC = 2
_NS = 16
_NW = _NC * _NS
_BPW = BATCH // _NW
_CB = 2048
_QBLK = 13
_Q = _QBLK * _CB

_FPW = _BPW // 2
_FGRP = 32


def _pack_pair(lo, hi):
    ul = lax.bitcast_convert_type(lo.astype(jnp.bfloat16), jnp.uint16)
    uh = lax.bitcast_convert_type(hi.astype(jnp.bfloat16), jnp.uint16)
    w = (uh.astype(jnp.uint32) << 16) | ul.astype(jnp.uint32)
    return lax.bitcast_convert_type(w, jnp.int32)


def _quad_body(x0_ref, x1_ref, x2_ref, x3_ref, out_ref):
    parts = []
    for r in (x0_ref, x1_ref, x2_ref, x3_ref):
        x = r[...]
        parts.append(_pack_pair(x[:32, :], x[32:, :]))
    out_ref[...] = jnp.concatenate(parts, axis=0).T


def _make_quad_table(ct):
    specs = [
        pl.BlockSpec(
            (FEATURE_DIM, _CB),
            functools.partial(
                lambda s, i: (0, jnp.minimum(i + _QBLK * s, 48)), s
            ),
        )
        for s in range(4)
    ]
    return pl.pallas_call(
        _quad_body,
        grid=(_QBLK,),
        in_specs=specs,
        out_specs=pl.BlockSpec((_CB, 128), lambda i: (i, 0)),
        out_shape=jax.ShapeDtypeStruct((_QBLK * _CB, 128), jnp.int32),
    )(ct, ct, ct, ct)


def _fpack_body(x_ref, out_ref):
    x = x_ref[...]
    n2 = jnp.sum(x * x, axis=0, keepdims=True)
    inv = lax.rsqrt(jnp.maximum(n2, 1e-24))
    xh = x * inv
    parts = []
    for a in range(_FGRP):
        xa = xh[:, a * _BPW:(a + 1) * _BPW]
        parts.append(
            jnp.concatenate([xa[:, :_FPW], xa[:, _FPW:]], axis=0).T
        )
    out_ref[...] = jnp.concatenate(parts, axis=0)


def _make_fpack(ft):
    return pl.pallas_call(
        _fpack_body,
        grid=(_NW // _FGRP,),
        in_specs=[pl.BlockSpec((FEATURE_DIM, _FGRP * _BPW), lambda i: (0, i))],
        out_specs=pl.BlockSpec((_FGRP * _FPW, 128), lambda i: (i, 0)),
        out_shape=jax.ShapeDtypeStruct((_NW * _FPW, 128), jnp.float32),
    )(ft)


def _sc_loss_body(labels_hbm, tab_hbm, fpack_hbm, out_hbm,
                  lab_v, jbuf, rows_v, fblk, accbuf, sem):
    wid = lax.axis_index("s") * _NC + lax.axis_index("c")
    base = wid * _BPW
    pltpu.sync_copy(labels_hbm.at[pl.ds(base, _BPW)], lab_v)
    pltpu.sync_copy(fpack_hbm.at[pl.ds(wid * _FPW, _FPW)], fblk)
    for g in range(_BPW // 16):
        lv = lab_v[pl.ds(16 * g, 16)]
        s = (
            jnp.where(lv >= _Q, 1, 0)
            + jnp.where(lv >= 2 * _Q, 1, 0)
            + jnp.where(lv >= 3 * _Q, 1, 0)
        ).astype(jnp.int32)
        jbuf[pl.ds(16 * g, 16)] = 4 * (lv - _Q * s) + s
    pltpu.async_copy(tab_hbm.at[jbuf], rows_v, sem).wait()

    def group(g, acc):
        f_base = 64 * (g // 16)
        for j in range(16):
            crow = 16 * g + j
            frow = 16 * (g % 16) + j
            for m in range(2):
                w = rows_v[crow, pl.ds(16 * m, 16)]
                wu = lax.bitcast_convert_type(w, jnp.uint32)
                clo = lax.bitcast_convert_type(wu << 16, jnp.float32)
                chi = lax.bitcast_convert_type(
                    wu & jnp.uint32(0xFFFF0000), jnp.float32)
                flo = fblk[frow, pl.ds(f_base + 16 * m, 16)]
                fhi = fblk[frow, pl.ds(f_base + 32 + 16 * m, 16)]
                dlo = flo - clo
                dhi = fhi - chi
                acc = acc + dlo * dlo
                acc = acc + dhi * dhi
        return acc

    acc = lax.fori_loop(0, _BPW // 16, group, jnp.zeros((16,), jnp.float32))
    accbuf[...] = acc
    pltpu.sync_copy(accbuf, out_hbm.at[wid])


@functools.cache
def _sc_loss():
    return pl.kernel(
        _sc_loss_body,
        out_type=jax.ShapeDtypeStruct((_NW, 16), jnp.float32),
        mesh=plsc.VectorSubcoreMesh(core_axis_name="c", subcore_axis_name="s"),
        scratch_types=[
            pltpu.VMEM((_BPW,), jnp.int32),
            pltpu.VMEM((_BPW,), jnp.int32),
            pltpu.VMEM((_BPW, 32), jnp.int32),
            pltpu.VMEM((_FPW, 128), jnp.float32),
            pltpu.VMEM((16,), jnp.float32),
            pltpu.SemaphoreType.DMA,
        ],
        compiler_params=pltpu.CompilerParams(use_tc_tiling_on_sc=False),
    )


def kernel(features, labels, centers):
    tab = _make_quad_table(centers.T)
    tab32 = jnp.reshape(tab, (4 * _QBLK * _CB, 32))
    fpack = _make_fpack(features.T)
    partials = _sc_loss()(labels.astype(jnp.int32), tab32, fpack)
    return jnp.sum(partials) * (0.5 / BATCH)


# speedup vs baseline: 1.0214x; 1.0214x over previous
import functools

import jax
import jax.numpy as jnp
from jax import lax
from jax.experimental import pallas as pl
from jax.experimental.pallas import tpu as pltpu
from jax.experimental.pallas import tpu_sc as plsc

CLS_NUM = 100000
FEATURE_DIM = 64
BATCH = 16384

_NC = 2
_NS = 16
_NW = _NC * _NS
_BPW = BATCH // _NW
_CB = 2048
_QBLK = 13
_Q = _QBLK * _CB

_FPW = _BPW // 2
_FGRP = 16


def _pack_pair(lo, hi):
    ul = lax.bitcast_convert_type(lo.astype(jnp.bfloat16), jnp.uint16)
    uh = lax.bitcast_convert_type(hi.astype(jnp.bfloat16), jnp.uint16)
    w = (uh.astype(jnp.uint32) << 16) | ul.astype(jnp.uint32)
    return lax.bitcast_convert_type(w, jnp.int32)


def _quad_body(x0_ref, x1_ref, x2_ref, x3_ref, out_ref):
    parts = []
    for r in (x0_ref, x1_ref, x2_ref, x3_ref):
        x = r[...]
        parts.append(_pack_pair(x[:32, :], x[32:, :]))
    out_ref[...] = jnp.concatenate(parts, axis=0).T


def _make_quad_table(ct):
    specs = [
        pl.BlockSpec(
            (FEATURE_DIM, _CB),
            functools.partial(
                lambda s, i: (0, jnp.minimum(i + _QBLK * s, 48)), s
            ),
        )
        for s in range(4)
    ]
    return pl.pallas_call(
        _quad_body,
        grid=(_QBLK,),
        in_specs=specs,
        out_specs=pl.BlockSpec((_CB, 128), lambda i: (i, 0)),
        out_shape=jax.ShapeDtypeStruct((_QBLK * _CB, 128), jnp.int32),
    )(ct, ct, ct, ct)


def _fpack_body(x_ref, out_ref):
    x = x_ref[...]
    n2 = jnp.sum(x * x, axis=0, keepdims=True)
    inv = lax.rsqrt(jnp.maximum(n2, 1e-24))
    xh = x * inv
    parts = []
    for a in range(_FGRP):
        xa = xh[:, a * _BPW:(a + 1) * _BPW]
        parts.append(
            jnp.concatenate([xa[:, :_FPW], xa[:, _FPW:]], axis=0).T
        )
    out_ref[...] = jnp.concatenate(parts, axis=0)


def _make_fpack(ft):
    return pl.pallas_call(
        _fpack_body,
        grid=(_NW // _FGRP,),
        in_specs=[pl.BlockSpec((FEATURE_DIM, _FGRP * _BPW), lambda i: (0, i))],
        out_specs=pl.BlockSpec((_FGRP * _FPW, 128), lambda i: (i, 0)),
        out_shape=jax.ShapeDtypeStruct((_NW * _FPW, 128), jnp.float32),
    )(ft)


def _sc_loss_body(labels_hbm, tab_hbm, fpack_hbm, out_hbm,
                  lab_v, jbuf, rows_v, fblk, accbuf, sem):
    wid = lax.axis_index("s") * _NC + lax.axis_index("c")
    base = wid * _BPW
    pltpu.sync_copy(labels_hbm.at[pl.ds(base, _BPW)], lab_v)
    pltpu.sync_copy(fpack_hbm.at[pl.ds(wid * _FPW, _FPW)], fblk)
    for g in range(_BPW // 16):
        lv = lab_v[pl.ds(16 * g, 16)]
        s = (
            jnp.where(lv >= _Q, 1, 0)
            + jnp.where(lv >= 2 * _Q, 1, 0)
            + jnp.where(lv >= 3 * _Q, 1, 0)
        ).astype(jnp.int32)
        jbuf[pl.ds(16 * g, 16)] = 4 * (lv - _Q * s) + s
    pltpu.async_copy(tab_hbm.at[jbuf], rows_v, sem).wait()

    def group(g, acc):
        f_base = 64 * (g // 16)
        for j in range(16):
            crow = 16 * g + j
            frow = 16 * (g % 16) + j
            for m in range(2):
                w = rows_v[crow, pl.ds(16 * m, 16)]
                wu = lax.bitcast_convert_type(w, jnp.uint32)
                clo = lax.bitcast_convert_type(wu << 16, jnp.float32)
                chi = lax.bitcast_convert_type(
                    wu & jnp.uint32(0xFFFF0000), jnp.float32)
                flo = fblk[frow, pl.ds(f_base + 16 * m, 16)]
                fhi = fblk[frow, pl.ds(f_base + 32 + 16 * m, 16)]
                dlo = flo - clo
                dhi = fhi - chi
                acc = acc + dlo * dlo
                acc = acc + dhi * dhi
        return acc

    acc = lax.fori_loop(0, _BPW // 16, group, jnp.zeros((16,), jnp.float32))
    accbuf[...] = acc
    pltpu.sync_copy(accbuf, out_hbm.at[wid])


@functools.cache
def _sc_loss():
    return pl.kernel(
        _sc_loss_body,
        out_type=jax.ShapeDtypeStruct((_NW, 16), jnp.float32),
        mesh=plsc.VectorSubcoreMesh(core_axis_name="c", subcore_axis_name="s"),
        scratch_types=[
            pltpu.VMEM((_BPW,), jnp.int32),
            pltpu.VMEM((_BPW,), jnp.int32),
            pltpu.VMEM((_BPW, 32), jnp.int32),
            pltpu.VMEM((_FPW, 128), jnp.float32),
            pltpu.VMEM((16,), jnp.float32),
            pltpu.SemaphoreType.DMA,
        ],
        compiler_params=pltpu.CompilerParams(use_tc_tiling_on_sc=False),
    )


def kernel(features, labels, centers):
    tab = _make_quad_table(centers.T)
    tab32 = jnp.reshape(tab, (4 * _QBLK * _CB, 32))
    fpack = _make_fpack(features.T)
    partials = _sc_loss()(labels.astype(jnp.int32), tab32, fpack)
    return jnp.sum(partials) * (0.5 / BATCH)
